# baseline (speedup 1.0000x reference)
import functools

import jax
import jax.numpy as jnp
from jax import lax
from jax.experimental import pallas as pl
from jax.experimental.pallas import tpu as pltpu
from jax.experimental.pallas import tpu_sc as plsc

B = 4096
HIDDEN = 128
GAMMA = 12.0
NC = 2
NS = 16
NW = NC * NS
BPW = B // NW
TB = 512


def _sc_gather(hidx_hbm, ridx_hbm, tidx_hbm,
               ex_hbm, ey_hbm, ez_hbm,
               rw_hbm, rx_hbm, ry_hbm, rz_hbm,
               o_hx, o_hy, o_hz, o_tx, o_ty, o_tz,
               o_rw, o_rx, o_ry, o_rz,
               hidx_v, ridx_v, tidx_v,
               buf_a, buf_b, gsem, wsem):
    wid = lax.axis_index("s") * NC + lax.axis_index("c")
    base = wid * BPW
    sl = pl.ds(base, BPW)
    pltpu.sync_copy(hidx_hbm.at[sl], hidx_v)
    pltpu.sync_copy(ridx_hbm.at[sl], ridx_v)
    pltpu.sync_copy(tidx_hbm.at[sl], tidx_v)

    jobs = [
        (ex_hbm, hidx_v, o_hx),
        (ey_hbm, hidx_v, o_hy),
        (ez_hbm, hidx_v, o_hz),
        (ex_hbm, tidx_v, o_tx),
        (ey_hbm, tidx_v, o_ty),
        (ez_hbm, tidx_v, o_tz),
        (rw_hbm, ridx_v, o_rw),
        (rx_hbm, ridx_v, o_rx),
        (ry_hbm, ridx_v, o_ry),
        (rz_hbm, ridx_v, o_rz),
    ]
    bufs = [buf_a, buf_b]
    pending = [None, None]
    for k, (tab, idx, out) in enumerate(jobs):
        b = k % 2
        if pending[b] is not None:
            pending[b].wait()
        pltpu.async_copy(tab.at[idx], bufs[b], gsem).wait()
        pending[b] = pltpu.async_copy(bufs[b], out.at[sl], wsem)
    pending[0].wait()
    pending[1].wait()


_row_sds = jax.ShapeDtypeStruct((B, HIDDEN), jnp.float32)
_sc_gather_call = functools.partial(
    pl.kernel,
    out_type=[_row_sds] * 10,
    mesh=plsc.VectorSubcoreMesh(core_axis_name="c", subcore_axis_name="s"),
    compiler_params=pltpu.CompilerParams(needs_layout_passes=False),
    scratch_types=[
        pltpu.VMEM((BPW,), jnp.int32),
        pltpu.VMEM((BPW,), jnp.int32),
        pltpu.VMEM((BPW,), jnp.int32),
        pltpu.VMEM((BPW, HIDDEN), jnp.float32),
        pltpu.VMEM((BPW, HIDDEN), jnp.float32),
        pltpu.SemaphoreType.DMA,
        pltpu.SemaphoreType.DMA,
    ],
)(_sc_gather)


def _tc_body(hx_r, hy_r, hz_r, tx_r, ty_r, tz_r,
             qw_r, qx_r, qy_r, qz_r,
             sc_o, s1_o, s2_o, adx_o):
    rw = qw_r[...]
    rx = qx_r[...]
    ry = qy_r[...]
    rz = qz_r[...]
    inv = lax.rsqrt(rw * rw + rx * rx + ry * ry + rz * rz)
    w = rw * inv
    x = rx * inv
    y = ry * inv
    z = rz * inv
    x2 = x + x
    y2 = y + y
    z2 = z + z
    xx = x2 * x
    yy = y2 * y
    zz = z2 * z
    xy = x2 * y
    xz = x2 * z
    yz = y2 * z
    xw = x2 * w
    yw = y2 * w
    zw = z2 * w
    hx = hx_r[...]
    hy = hy_r[...]
    hz = hz_r[...]
    dx = (1.0 - yy - zz) * hx + (xy - zw) * hy + (xz + yw) * hz - tx_r[...]
    dy = (xy + zw) * hx + (1.0 - xx - zz) * hy + (yz - xw) * hz - ty_r[...]
    dz = (xz - yw) * hx + (yz + xw) * hy + (1.0 - xx - yy) * hz - tz_r[...]
    s1e = lax.sqrt(dx * dx + dy * dy + dz * dz)
    adx_o[...] = jnp.abs(dx)
    s1m = jnp.sum(s1e, axis=1, keepdims=True) * (1.0 / HIDDEN)
    s1_o[...] = s1m
    s2_o[...] = s1m
    sc_o[...] = GAMMA - s1m


_col_spec = pl.BlockSpec((TB, 1), lambda i: (i, 0))
_row_spec = pl.BlockSpec((TB, HIDDEN), lambda i: (i, 0))
_tc_call = pl.pallas_call(
    _tc_body,
    grid=(B // TB,),
    in_specs=[_row_spec] * 10,
    out_specs=[_col_spec, _col_spec, _col_spec, _row_spec],
    out_shape=[
        jax.ShapeDtypeStruct((B, 1), jnp.float32),
        jax.ShapeDtypeStruct((B, 1), jnp.float32),
        jax.ShapeDtypeStruct((B, 1), jnp.float32),
        jax.ShapeDtypeStruct((B, HIDDEN), jnp.float32),
    ],
)


def kernel(sample, entity_x, entity_y, entity_z,
           relation_w, relation_x, relation_y, relation_z):
    h_idx = sample[:, 0]
    r_idx = sample[:, 1]
    t_idx = sample[:, 2]
    rows = _sc_gather_call(
        h_idx, r_idx, t_idx,
        entity_x, entity_y, entity_z,
        relation_w, relation_x, relation_y, relation_z,
    )
    score, s1, s2, adx = _tc_call(*rows)
    return score, s1, s2, adx[:, None, :]


# speedup vs baseline: 1.3458x; 1.3458x over previous
import functools

import jax
import jax.numpy as jnp
from jax import lax
from jax.experimental import pallas as pl
from jax.experimental.pallas import tpu as pltpu
from jax.experimental.pallas import tpu_sc as plsc

B = 4096
HIDDEN = 128
GAMMA = 12.0
NC = 2
NS = 16
NW = NC * NS
BPW = B // NW
CH = 32
NCH = BPW // CH
ND = HIDDEN // 16
TINY = 1e-35


def _rsqrt1(s):
    i = lax.bitcast_convert_type(s, jnp.int32)
    i = jnp.int32(0x5F3759DF) - (i >> 1)
    y = lax.bitcast_convert_type(i, jnp.float32)
    y = y * (1.5 - (0.5 * s) * y * y)
    return y


def _sc_body(*args):
    (hidx_hbm, ridx_hbm, tidx_hbm,
     ex_hbm, ey_hbm, ez_hbm,
     rw_hbm, rx_hbm, ry_hbm, rz_hbm,
     score_hbm, s1_hbm, s2_hbm, adx_hbm) = args[:14]
    hidx_v, ridx_v, tidx_v = args[14:17]
    gsets = (args[17:27], args[27:37])
    adx_bufs = args[37:39]
    sc_v, s1_v, s2_v = args[39:42]
    gsem, wsem = args[42:44]

    wid = lax.axis_index("s") * NC + lax.axis_index("c")
    base = wid * BPW
    pltpu.sync_copy(hidx_hbm.at[pl.ds(base, BPW)], hidx_v)
    pltpu.sync_copy(ridx_hbm.at[pl.ds(base, BPW)], ridx_v)
    pltpu.sync_copy(tidx_hbm.at[pl.ds(base, BPW)], tidx_v)

    def issue(c):
        bufs = gsets[c % 2]
        hs = hidx_v.at[pl.ds(c * CH, CH)]
        ts = tidx_v.at[pl.ds(c * CH, CH)]
        rs = ridx_v.at[pl.ds(c * CH, CH)]
        return [
            pltpu.async_copy(ex_hbm.at[hs], bufs[0], gsem),
            pltpu.async_copy(ey_hbm.at[hs], bufs[1], gsem),
            pltpu.async_copy(ez_hbm.at[hs], bufs[2], gsem),
            pltpu.async_copy(ex_hbm.at[ts], bufs[3], gsem),
            pltpu.async_copy(ey_hbm.at[ts], bufs[4], gsem),
            pltpu.async_copy(ez_hbm.at[ts], bufs[5], gsem),
            pltpu.async_copy(rw_hbm.at[rs], bufs[6], gsem),
            pltpu.async_copy(rx_hbm.at[rs], bufs[7], gsem),
            pltpu.async_copy(ry_hbm.at[rs], bufs[8], gsem),
            pltpu.async_copy(rz_hbm.at[rs], bufs[9], gsem),
        ]

    inflight = {0: issue(0), 1: issue(1)}
    pending_wb = [None, None]

    for c in range(NCH):
        for cp in inflight.pop(c):
            cp.wait()
        if pending_wb[c % 2] is not None:
            pending_wb[c % 2].wait()
        hx_v, hy_v, hz_v, tx_v, ty_v, tz_v, qw_v, qx_v, qy_v, qz_v = gsets[c % 2]
        adx_v = adx_bufs[c % 2]

        def row(r, carry):
            p_sc, p_s1 = carry
            a1 = jnp.zeros((16,), jnp.float32)
            for d in range(ND):
                ds16 = pl.ds(d * 16, 16)
                rw = qw_v[r, ds16]
                rx = qx_v[r, ds16]
                ry = qy_v[r, ds16]
                rz = qz_v[r, ds16]
                hx = hx_v[r, ds16]
                hy = hy_v[r, ds16]
                hz = hz_v[r, ds16]
                tx = tx_v[r, ds16]
                ty = ty_v[r, ds16]
                tz = tz_v[r, ds16]
                s = rw * rw + rx * rx + ry * ry + rz * rz
                k = 2.0 / jnp.maximum(s, TINY)
                kx = k * rx
                ky = k * ry
                kz = k * rz
                xx = kx * rx
                xy = kx * ry
                xz = kx * rz
                xw = kx * rw
                yy = ky * ry
                yz = ky * rz
                yw = ky * rw
                zz = kz * rz
                zw = kz * rw
                a11 = 1.0 - yy - zz
                a12 = xy - zw
                a13 = xz + yw
                a21 = xy + zw
                a22 = 1.0 - xx - zz
                a23 = yz - xw
                a31 = xz - yw
                a32 = yz + xw
                a33 = 1.0 - xx - yy
                dx = a11 * hx + a12 * hy + a13 * hz - tx
                dy = a21 * hx + a22 * hy + a23 * hz - ty
                dz = a31 * hx + a32 * hy + a33 * hz - tz
                q1 = dx * dx + dy * dy + dz * dz
                a1 = a1 + q1 * _rsqrt1(jnp.maximum(q1, TINY))
                adx_v[r, ds16] = jnp.abs(dx)
            s1m = jnp.sum(a1) * (1.0 / HIDDEN)
            lane = r & 15
            m = lax.iota(jnp.int32, 16) == lane
            p_s1 = jnp.where(m, s1m, p_s1)
            p_sc = jnp.where(m, GAMMA - s1m, p_sc)

            @pl.when(lane == 15)
            def _flush():
                g = pl.multiple_of(c * CH + r - 15, 16)
                sc_v[pl.ds(g, 16)] = p_sc
                s1_v[pl.ds(g, 16)] = p_s1
                s2_v[pl.ds(g, 16)] = p_s1

            return p_sc, p_s1

        zero16 = jnp.zeros((16,), jnp.float32)
        lax.fori_loop(0, CH, row, (zero16, zero16))

        pending_wb[c % 2] = pltpu.async_copy(
            adx_v, adx_hbm.at[pl.ds(base + c * CH, CH)], wsem)
        if c + 2 < NCH:
            inflight[c + 2] = issue(c + 2)

    pending_wb[0].wait()
    pending_wb[1].wait()
    pltpu.sync_copy(sc_v, score_hbm.at[pl.ds(base, BPW)])
    pltpu.sync_copy(s1_v, s1_hbm.at[pl.ds(base, BPW)])
    pltpu.sync_copy(s2_v, s2_hbm.at[pl.ds(base, BPW)])


_sc_call = functools.partial(
    pl.kernel,
    out_type=[
        jax.ShapeDtypeStruct((B,), jnp.float32),
        jax.ShapeDtypeStruct((B,), jnp.float32),
        jax.ShapeDtypeStruct((B,), jnp.float32),
        jax.ShapeDtypeStruct((B, HIDDEN), jnp.float32),
    ],
    mesh=plsc.VectorSubcoreMesh(core_axis_name="c", subcore_axis_name="s"),
    compiler_params=pltpu.CompilerParams(needs_layout_passes=False),
    scratch_types=(
        [pltpu.VMEM((BPW,), jnp.int32)] * 3
        + [pltpu.VMEM((CH, HIDDEN), jnp.float32)] * 20
        + [pltpu.VMEM((CH, HIDDEN), jnp.float32)] * 2
        + [pltpu.VMEM((BPW,), jnp.float32)] * 3
        + [pltpu.SemaphoreType.DMA, pltpu.SemaphoreType.DMA]
    ),
)(_sc_body)


def kernel(sample, entity_x, entity_y, entity_z,
           relation_w, relation_x, relation_y, relation_z):
    h_idx = sample[:, 0]
    r_idx = sample[:, 1]
    t_idx = sample[:, 2]
    score, s1, s2, adx = _sc_call(
        h_idx, r_idx, t_idx,
        entity_x, entity_y, entity_z,
        relation_w, relation_x, relation_y, relation_z,
    )
    return score[:, None], s1[:, None], s2[:, None], adx[:, None, :]
